# baseline (speedup 1.0000x reference)
import jax
import jax.numpy as jnp
from jax.experimental import pallas as pl

_B = 4
_L = 1024
_H = 4
_D = 32
_BS = 32
_S = 4
_T = _B * _L
_NB = _L // _BS
_SCALE = _D ** (-0.5)


def _silu(x):
    return x * jax.nn.sigmoid(x)


def _fwd(q_ref, k_ref, v_ref, gc_ref, gs_ref, o_ref):
    blk_of_col = jax.lax.broadcasted_iota(jnp.int32, (_NB, _L), 1) // _BS
    blk_row = jax.lax.broadcasted_iota(jnp.int32, (_NB, _L), 0)
    expand = (blk_of_col == blk_row).astype(jnp.float32)

    row = jax.lax.broadcasted_iota(jnp.int32, (_L, _NB), 0)
    col = jax.lax.broadcasted_iota(jnp.int32, (_L, _NB), 1)
    blk_causal = (row // _BS) >= col
    rowl = jax.lax.broadcasted_iota(jnp.int32, (_L, _L), 0)
    coll = jax.lax.broadcasted_iota(jnp.int32, (_L, _L), 1)
    tok_causal = coll <= rowl
    neginf = jnp.float32(-jnp.inf)

    for h in range(_H):
        qt = q_ref[0, :, h, :]
        kk = k_ref[0, :, h, :]
        vv = v_ref[0, :, h, :]
        gc = gc_ref[0, :, h][:, None]
        gs = gs_ref[0, :, h][:, None]

        k_cmp = kk.reshape(_NB, _BS, _D).sum(axis=1) * (1.0 / _BS)
        v_cmp = vv.reshape(_NB, _BS, _D).sum(axis=1) * (1.0 / _BS)

        s_cmp = jax.lax.dot_general(
            qt, k_cmp, (((1,), (1,)), ((), ())),
            preferred_element_type=jnp.float32) * _SCALE
        p_cmp = jnp.where(blk_causal, _silu(s_cmp), 0.0)
        o_cmp = jnp.dot(p_cmp, v_cmp,
                        preferred_element_type=jnp.float32) * gc

        work = jnp.where(blk_causal, s_cmp, neginf)
        sel = jnp.zeros((_L, _NB), dtype=jnp.bool_)
        for _ in range(_S):
            m = jnp.max(work, axis=1, keepdims=True)
            ismax = jnp.logical_and(work == m, jnp.logical_not(sel))
            cand = jnp.where(ismax, col, _NB)
            mi = jnp.min(cand, axis=1, keepdims=True)
            pick = col == mi
            sel = jnp.logical_or(sel, pick)
            work = jnp.where(pick, neginf, work)

        s_full = jax.lax.dot_general(
            qt, kk, (((1,), (1,)), ((), ())),
            preferred_element_type=jnp.float32) * _SCALE
        selm = jnp.dot(sel.astype(jnp.float32), expand,
                       preferred_element_type=jnp.float32)
        keep = jnp.logical_and(selm > 0.5, tok_causal)
        p = jnp.where(keep, _silu(s_full), 0.0)
        o_slc = jnp.dot(p, vv, preferred_element_type=jnp.float32) * gs

        o_ref[0, :, h, :] = o_cmp + o_slc


def _run(qh, kh, vh, gc, gs, interpret=False):
    return pl.pallas_call(
        _fwd,
        grid=(_B,),
        in_specs=[
            pl.BlockSpec((1, _L, _H, _D), lambda b: (b, 0, 0, 0)),
            pl.BlockSpec((1, _L, _H, _D), lambda b: (b, 0, 0, 0)),
            pl.BlockSpec((1, _L, _H, _D), lambda b: (b, 0, 0, 0)),
            pl.BlockSpec((1, _L, _H), lambda b: (b, 0, 0)),
            pl.BlockSpec((1, _L, _H), lambda b: (b, 0, 0)),
        ],
        out_specs=pl.BlockSpec((1, _L, _H, _D), lambda b: (b, 0, 0, 0)),
        out_shape=jax.ShapeDtypeStruct((_B, _L, _H, _D), jnp.float32),
        interpret=interpret,
    )(qh, kh, vh, gc, gs)


def kernel(q, k, v, g_cmp, g_slc, x_offsets):
    del x_offsets
    qh = q.reshape(_B, _L, _H, _D)
    kh = k.reshape(_B, _L, _H, _D)
    vh = v.reshape(_B, _L, _H, _D)
    gc = g_cmp.reshape(_B, _L, _H)
    gs = g_slc.reshape(_B, _L, _H)
    return _run(qh, kh, vh, gc, gs).reshape(_T, _H, _D)
